# baseline (speedup 1.0000x reference)
import functools

import jax
import jax.numpy as jnp
from jax import lax
from jax.experimental import pallas as pl
from jax.experimental.pallas import tpu as pltpu
from jax.experimental.pallas import tpu_sc as plsc

N = 10000
E = 320000
C1 = 128
COUT = 40
DEGC = 128

NC = 2
NS = 16
NW = NC * NS
EW = E // NW
K = 80
CHUNKS = EW // K
KA = 128
CA = 79
EWP = KA * CA
DUMP = N
RPT = 632
NP = NS * RPT
ZR = 8

R = 1000
GRID = N // R

_mesh = functools.partial(
    plsc.VectorSubcoreMesh, core_axis_name="c", subcore_axis_name="s"
)


def _fill(ref, val):
  rows, cols = ref.shape

  def row(r, _):
    def col(q, __):
      ref[r, pl.ds(q * 16, 16)] = jnp.full((16,), val, jnp.float32)
      return __
    return lax.fori_loop(0, cols // 16, col, _)

  lax.fori_loop(0, rows, row, 0)


def _make_deg_kernel():
  @functools.partial(
      pl.kernel,
      out_type=jax.ShapeDtypeStruct((NC * NP, DEGC), jnp.float32),
      mesh=_mesh(),
      scratch_types=[
          pltpu.VMEM((CHUNKS, K), jnp.int32),
          pltpu.VMEM((K, DEGC), jnp.float32),
          pltpu.VMEM((ZR, DEGC), jnp.float32),
          pltpu.VMEM_SHARED((NP, DEGC), jnp.float32),
      ],
  )
  def deg_kernel(dst_hbm, out_hbm, dstv, ones_v, zbuf, accum):
    c = lax.axis_index("c")
    s = lax.axis_index("s")
    w = c * NS + s
    _fill(zbuf, 0.0)
    _fill(ones_v, 1.0)
    for k in range(RPT // ZR):
      pltpu.sync_copy(zbuf, accum.at[pl.ds(s * RPT + k * ZR, ZR)])
    plsc.subcore_barrier()
    pltpu.sync_copy(dst_hbm.at[w], dstv)

    def chunk(j, carry):
      pltpu.sync_copy(ones_v, accum.at[dstv.at[j]], add=True)
      return carry

    lax.fori_loop(0, CHUNKS, chunk, 0)
    plsc.subcore_barrier()
    pltpu.sync_copy(
        accum.at[pl.ds(s * RPT, RPT)],
        out_hbm.at[pl.ds(c * NP + s * RPT, RPT)],
    )

  return deg_kernel


def _make_agg_kernel(C):
  @functools.partial(
      pl.kernel,
      out_type=jax.ShapeDtypeStruct((NC * NP, C), jnp.float32),
      mesh=_mesh(),
      scratch_types=[
          pltpu.VMEM((2, KA), jnp.int32),
          pltpu.VMEM((2, KA), jnp.int32),
          pltpu.VMEM((KA, C), jnp.float32),
          pltpu.VMEM((KA, C), jnp.float32),
          pltpu.VMEM((ZR, C), jnp.float32),
          pltpu.VMEM_SHARED((NP, C), jnp.float32),
          pltpu.SemaphoreType.DMA,
          pltpu.SemaphoreType.DMA,
          pltpu.SemaphoreType.DMA,
          pltpu.SemaphoreType.DMA,
      ],
  )
  def agg_kernel(y_hbm, eidx_hbm, out_hbm, idx0, idx1, rows0, rows1,
                 zbuf, accum, semg0, semg1, semi0, semi1):
    c = lax.axis_index("c")
    s = lax.axis_index("s")
    w = c * NS + s
    _fill(zbuf, 0.0)
    for k in range(RPT // ZR):
      pltpu.sync_copy(zbuf, accum.at[pl.ds(s * RPT + k * ZR, ZR)])
    plsc.subcore_barrier()

    bufs = ((idx0, rows0, semg0, semi0), (idx1, rows1, semg1, semi1))

    def load_idx(j, p, sync):
      ib, _, _, si = bufs[p]
      if sync:
        pltpu.sync_copy(eidx_hbm.at[w, j], ib)
      else:
        pltpu.async_copy(eidx_hbm.at[w, j], ib, si)

    def wait_idx(p):
      ib, _, _, si = bufs[p]
      pltpu.make_async_copy(eidx_hbm.at[w, 0], ib, si).wait()

    def start_gather(p):
      ib, rb, sg, _ = bufs[p]
      pltpu.async_copy(y_hbm.at[ib.at[0]], rb, sg)

    def wait_gather(p):
      ib, rb, sg, _ = bufs[p]
      pltpu.make_async_copy(y_hbm.at[ib.at[0]], rb, sg).wait()

    def scatter(p):
      ib, rb, _, _ = bufs[p]
      pltpu.sync_copy(rb, accum.at[ib.at[1]], add=True)

    load_idx(0, 0, True)
    start_gather(0)
    load_idx(1, 1, True)
    start_gather(1)

    def pair(i, carry):
      j0 = 2 * i
      wait_gather(0)
      scatter(0)
      load_idx(j0 + 2, 0, False)
      wait_gather(1)
      scatter(1)
      wait_idx(0)
      start_gather(0)
      load_idx(j0 + 3, 1, False)
      wait_idx(1)
      start_gather(1)
      return carry

    lax.fori_loop(0, (CA - 3) // 2, pair, 0)
    wait_gather(0)
    scatter(0)
    load_idx(CA - 1, 0, False)
    wait_gather(1)
    scatter(1)
    wait_idx(0)
    start_gather(0)
    wait_gather(0)
    scatter(0)
    plsc.subcore_barrier()
    pltpu.sync_copy(
        accum.at[pl.ds(s * RPT, RPT)],
        out_hbm.at[pl.ds(c * NP + s * RPT, RPT)],
    )

  return agg_kernel


_deg = _make_deg_kernel()
_agg = _make_agg_kernel(C1)


def _tc1_body(x_ref, w1_ref, d0_ref, d1_ref, y1_ref, dinv_ref):
  deg = 1.0 + d0_ref[...] + d1_ref[...]
  dinv = lax.rsqrt(deg)
  xw = jnp.dot(x_ref[...], w1_ref[...], preferred_element_type=jnp.float32)
  y1_ref[...] = xw * dinv
  dinv_ref[...] = dinv


def _tc2_body(p0_ref, p1_ref, y1_ref, dinv_ref, b1_ref, w2_ref, y2_ref):
  dinv = dinv_ref[...]
  h1 = dinv * (p0_ref[...] + p1_ref[...] + y1_ref[...]) + b1_ref[...]
  t = jnp.dot(h1, w2_ref[...], preferred_element_type=jnp.float32)
  y2_ref[...] = t * dinv


def _tc3_body(q0_ref, q1_ref, y2_ref, dinv_ref, b2_ref, linw_ref, linb_ref,
              out_ref):
  h2 = dinv_ref[...] * (q0_ref[...] + q1_ref[...] + y2_ref[...]) + b2_ref[...]
  logits = jnp.dot(h2, linw_ref[...],
                   preferred_element_type=jnp.float32) + linb_ref[...]
  m = jnp.max(logits, axis=1, keepdims=True)
  lse = jnp.log(jnp.sum(jnp.exp(logits - m), axis=1, keepdims=True)) + m
  out_ref[...] = logits - lse


def _row_spec(cols):
  return pl.BlockSpec((R, cols), lambda i: (i, 0))


def _full_spec(rows, cols):
  return pl.BlockSpec((rows, cols), lambda i: (0, 0))


@jax.jit
def kernel(x, edge_index, W1, b1, W2, b2, lin_W, lin_b):
  ei = edge_index.astype(jnp.int32)
  dst3 = ei[1].reshape(NW, CHUNKS, K)
  srcp = jnp.pad(ei[0].reshape(NW, EW), ((0, 0), (0, EWP - EW)),
                 constant_values=0).reshape(NW, CA, KA)
  dstp = jnp.pad(ei[1].reshape(NW, EW), ((0, 0), (0, EWP - EW)),
                 constant_values=DUMP).reshape(NW, CA, KA)
  eidx = jnp.stack([srcp, dstp], axis=2)

  degp = _deg(dst3)
  d0 = degp[:N, 0:1]
  d1 = degp[NP:NP + N, 0:1]

  y1, dinv = pl.pallas_call(
      _tc1_body,
      grid=(GRID,),
      in_specs=[
          _row_spec(C1),
          _full_spec(C1, C1),
          _row_spec(1),
          _row_spec(1),
      ],
      out_specs=[_row_spec(C1), _row_spec(1)],
      out_shape=[
          jax.ShapeDtypeStruct((N, C1), jnp.float32),
          jax.ShapeDtypeStruct((N, 1), jnp.float32),
      ],
  )(x, W1, d0, d1)

  p = _agg(y1, eidx)

  y2 = pl.pallas_call(
      _tc2_body,
      grid=(GRID,),
      in_specs=[
          _row_spec(C1),
          _row_spec(C1),
          _row_spec(C1),
          _row_spec(1),
          _full_spec(1, C1),
          _full_spec(C1, C1),
      ],
      out_specs=[_row_spec(C1)],
      out_shape=[jax.ShapeDtypeStruct((N, C1), jnp.float32)],
  )(p[:N], p[NP:NP + N], y1, dinv, b1.reshape(1, C1), W2)[0]

  q = _agg(y2, eidx)

  out = pl.pallas_call(
      _tc3_body,
      grid=(GRID,),
      in_specs=[
          _row_spec(C1),
          _row_spec(C1),
          _row_spec(C1),
          _row_spec(1),
          _full_spec(1, C1),
          _full_spec(C1, COUT),
          _full_spec(1, COUT),
      ],
      out_specs=[_row_spec(COUT)],
      out_shape=[jax.ShapeDtypeStruct((N, COUT), jnp.float32)],
  )(q[:N], q[NP:NP + N], y2, dinv, b2.reshape(1, C1), lin_W,
    lin_b.reshape(1, COUT))[0]

  return out


# speedup vs baseline: 1.3079x; 1.3079x over previous
import functools

import jax
import jax.numpy as jnp
from jax import lax
from jax.experimental import pallas as pl
from jax.experimental.pallas import tpu as pltpu
from jax.experimental.pallas import tpu_sc as plsc

N = 10000
E = 320000
C1 = 128
COUT = 40

NC = 2
NS = 16
NW = NC * NS
EW = E // NW
KB = 80
CB = EW // KB
PSHIFT = 14
RPT = 632
NP = NS * RPT

HB = 16384
HT = HB // NS

R = 1000
GRID = N // R

_mesh = functools.partial(
    plsc.VectorSubcoreMesh, core_axis_name="c", subcore_axis_name="s"
)


def _make_deg_kernel():
  @functools.partial(
      pl.kernel,
      out_type=jax.ShapeDtypeStruct((NC, HB), jnp.float32),
      mesh=_mesh(),
      compiler_params=pltpu.CompilerParams(needs_layout_passes=False),
      scratch_types=[
          pltpu.VMEM((EW,), jnp.int32),
          pltpu.VMEM((HB,), jnp.float32),
          pltpu.VMEM((HT,), jnp.float32),
          pltpu.VMEM((HT,), jnp.float32),
          pltpu.VMEM_SHARED((NS, HB), jnp.float32),
      ],
  )
  def deg_kernel(eidx_hbm, out_hbm, eidxv, hist, rdbuf, sumbuf, shared):
    c = lax.axis_index("c")
    s = lax.axis_index("s")
    w = c * NS + s

    def zchunk(r, carry):
      hist[pl.ds(r * 16, 16)] = jnp.zeros((16,), jnp.float32)
      return carry

    lax.fori_loop(0, HB // 16, zchunk, 0)
    pltpu.sync_copy(eidx_hbm.at[w], eidxv)
    ones = jnp.ones((16,), jnp.float32)
    shift = jnp.full((16,), PSHIFT, jnp.int32)

    def chunk(i, carry):
      v = eidxv[pl.ds(i * 16, 16)]
      plsc.addupdate_scatter(hist, (lax.shift_right_logical(v, shift),),
                             ones)
      return carry

    lax.fori_loop(0, EW // 16, chunk, 0)
    pltpu.sync_copy(hist, shared.at[s])
    plsc.subcore_barrier()

    pltpu.sync_copy(shared.at[0, pl.ds(s * HT, HT)], sumbuf)
    for t in range(1, NS):
      pltpu.sync_copy(shared.at[t, pl.ds(s * HT, HT)], rdbuf)
      for q in range(HT // 16):
        sumbuf[pl.ds(q * 16, 16)] = (
            sumbuf[pl.ds(q * 16, 16)] + rdbuf[pl.ds(q * 16, 16)])
    pltpu.sync_copy(sumbuf, out_hbm.at[c, pl.ds(s * HT, HT)])

  return deg_kernel


def _make_agg_kernel(C):
  @functools.partial(
      pl.kernel,
      out_type=jax.ShapeDtypeStruct((NC * NP, C), jnp.float32),
      mesh=_mesh(),
      scratch_types=[
          pltpu.VMEM((EW,), jnp.int32),
          pltpu.VMEM((KB,), jnp.int32),
          pltpu.VMEM((KB,), jnp.int32),
          pltpu.VMEM((KB, C), jnp.float32),
          pltpu.VMEM_SHARED((NP, C), jnp.float32),
      ],
  )
  def agg_kernel(y_hbm, eidx_hbm, zero_hbm, out_hbm, eidxv, srcb, dstb,
                 rows, accum):
    c = lax.axis_index("c")
    s = lax.axis_index("s")
    w = c * NS + s
    pltpu.sync_copy(zero_hbm, accum.at[pl.ds(s * RPT, RPT)])
    pltpu.sync_copy(eidx_hbm.at[w], eidxv)
    plsc.subcore_barrier()

    mask = jnp.full((16,), (1 << PSHIFT) - 1, jnp.int32)
    shift = jnp.full((16,), PSHIFT, jnp.int32)

    def chunk(j, carry):
      for q in range(KB // 16):
        v = eidxv[pl.ds(j * KB + q * 16, 16)]
        srcb[pl.ds(q * 16, 16)] = lax.bitwise_and(v, mask)
        dstb[pl.ds(q * 16, 16)] = lax.shift_right_logical(v, shift)
      pltpu.sync_copy(y_hbm.at[srcb], rows)
      pltpu.sync_copy(rows, accum.at[dstb], add=True)
      return carry

    lax.fori_loop(0, CB, chunk, 0)
    plsc.subcore_barrier()
    pltpu.sync_copy(
        accum.at[pl.ds(s * RPT, RPT)],
        out_hbm.at[pl.ds(c * NP + s * RPT, RPT)],
    )

  return agg_kernel


_deg = _make_deg_kernel()
_agg = _make_agg_kernel(C1)


def _tc0_body(x_ref, w1_ref, xw_ref):
  xw_ref[...] = jnp.dot(x_ref[...], w1_ref[...],
                        preferred_element_type=jnp.float32)


def _tc1_body(xw_ref, d0_ref, d1_ref, y1_ref, dinv_ref):
  deg = 1.0 + d0_ref[...] + d1_ref[...]
  dinv = lax.rsqrt(deg)
  y1_ref[...] = xw_ref[...] * dinv
  dinv_ref[...] = dinv


def _tc2_body(p0_ref, p1_ref, y1_ref, dinv_ref, b1_ref, w2_ref, y2_ref):
  dinv = dinv_ref[...]
  h1 = dinv * (p0_ref[...] + p1_ref[...] + y1_ref[...]) + b1_ref[...]
  t = jnp.dot(h1, w2_ref[...], preferred_element_type=jnp.float32)
  y2_ref[...] = t * dinv


def _tc3_body(q0_ref, q1_ref, y2_ref, dinv_ref, b2_ref, linw_ref, linb_ref,
              out_ref):
  h2 = dinv_ref[...] * (q0_ref[...] + q1_ref[...] + y2_ref[...]) + b2_ref[...]
  logits = jnp.dot(h2, linw_ref[...],
                   preferred_element_type=jnp.float32) + linb_ref[...]
  m = jnp.max(logits, axis=1, keepdims=True)
  lse = jnp.log(jnp.sum(jnp.exp(logits - m), axis=1, keepdims=True)) + m
  out_ref[...] = logits - lse


def _row_spec(cols):
  return pl.BlockSpec((R, cols), lambda i: (i, 0))


def _full_spec(rows, cols):
  return pl.BlockSpec((rows, cols), lambda i: (0, 0))


@jax.jit
def kernel(x, edge_index, W1, b1, W2, b2, lin_W, lin_b):
  ei = edge_index.astype(jnp.int32)
  eidx = (ei[0] + ei[1] * (1 << PSHIFT)).reshape(NW, EW)
  zrows = jnp.zeros((RPT, C1), jnp.float32)

  degp = _deg(eidx)
  d0 = degp[0, :N].reshape(N, 1)
  d1 = degp[1, :N].reshape(N, 1)

  xw = pl.pallas_call(
      _tc0_body,
      grid=(GRID,),
      in_specs=[_row_spec(C1), _full_spec(C1, C1)],
      out_specs=[_row_spec(C1)],
      out_shape=[jax.ShapeDtypeStruct((N, C1), jnp.float32)],
  )(x, W1)[0]

  y1, dinv = pl.pallas_call(
      _tc1_body,
      grid=(GRID,),
      in_specs=[
          _row_spec(C1),
          _row_spec(1),
          _row_spec(1),
      ],
      out_specs=[_row_spec(C1), _row_spec(1)],
      out_shape=[
          jax.ShapeDtypeStruct((N, C1), jnp.float32),
          jax.ShapeDtypeStruct((N, 1), jnp.float32),
      ],
  )(xw, d0, d1)

  p = _agg(y1, eidx, zrows)

  y2 = pl.pallas_call(
      _tc2_body,
      grid=(GRID,),
      in_specs=[
          _row_spec(C1),
          _row_spec(C1),
          _row_spec(C1),
          _row_spec(1),
          _full_spec(1, C1),
          _full_spec(C1, C1),
      ],
      out_specs=[_row_spec(C1)],
      out_shape=[jax.ShapeDtypeStruct((N, C1), jnp.float32)],
  )(p[:N], p[NP:NP + N], y1, dinv, b1.reshape(1, C1), W2)[0]

  q = _agg(y2, eidx, zrows)

  out = pl.pallas_call(
      _tc3_body,
      grid=(GRID,),
      in_specs=[
          _row_spec(C1),
          _row_spec(C1),
          _row_spec(C1),
          _row_spec(1),
          _full_spec(1, C1),
          _full_spec(C1, COUT),
          _full_spec(1, COUT),
      ],
      out_specs=[_row_spec(COUT)],
      out_shape=[jax.ShapeDtypeStruct((N, COUT), jnp.float32)],
  )(q[:N], q[NP:NP + N], y2, dinv, b2.reshape(1, C1), lin_W,
    lin_b.reshape(1, COUT))[0]

  return out
